# Initial kernel scaffold; baseline (speedup 1.0000x reference)
import jax
import jax.numpy as jnp
from jax.experimental import pallas as pl


def kernel(x, edge_index, bn_gamma, bn_beta, W1, b1, ln_gamma, ln_beta, W2, b2):
    raise NotImplementedError("write your pallas kernel here")


# speedup vs baseline: 13.0099x; 13.0099x over previous
import functools

import jax
import jax.numpy as jnp
from jax import lax
from jax.experimental import pallas as pl
from jax.experimental.pallas import tpu as pltpu
from jax.experimental.pallas import tpu_sc as plsc

N = 10000
E = 320000
D_IN = 128
D_H = 96
D_OUT = 64
EPS = 1e-5

NC = 2
NS = 16
NW = NC * NS
C = 128
NCHUNK = 80
EPW = NCHUNK * C
E_PAD = NW * EPW
N_PAD = 10240
RPT = N_PAD // NS


def _mesh():
    return plsc.VectorSubcoreMesh(core_axis_name="c", subcore_axis_name="s")


def _sc_deg(dst3, ones_c, zeros_rpt):
    @functools.partial(
        pl.kernel,
        mesh=_mesh(),
        compiler_params=pltpu.CompilerParams(use_tc_tiling_on_sc=False),
        out_type=jax.ShapeDtypeStruct((NC, N_PAD), jnp.float32),
        scratch_types=[
            pltpu.VMEM((NCHUNK, C), jnp.int32),
            pltpu.VMEM((C,), jnp.float32),
            pltpu.VMEM((RPT,), jnp.float32),
            pltpu.VMEM_SHARED((N_PAD,), jnp.float32),
        ],
    )
    def k(dst_h, ones_h, zeros_h, out_h, dst_v, ones_v, bounce_v, acc_sh):
        cid = lax.axis_index("c")
        sid = lax.axis_index("s")
        wid = sid * NC + cid
        pltpu.sync_copy(ones_h, ones_v)
        pltpu.sync_copy(zeros_h, bounce_v)
        pltpu.sync_copy(bounce_v, acc_sh.at[pl.ds(sid * RPT, RPT)])
        pltpu.sync_copy(dst_h.at[wid], dst_v)
        plsc.subcore_barrier()

        def body(j, carry):
            pltpu.sync_copy(ones_v, acc_sh.at[dst_v.at[j]], add=True)
            return carry

        lax.fori_loop(0, NCHUNK, body, 0)
        plsc.subcore_barrier()
        pltpu.sync_copy(acc_sh.at[pl.ds(sid * RPT, RPT)], bounce_v)
        pltpu.sync_copy(bounce_v, out_h.at[cid, pl.ds(sid * RPT, RPT)])

    return k(dst3, ones_c, zeros_rpt)


def _sc_agg(h, src3, dst3, zeros_blk, d):
    @functools.partial(
        pl.kernel,
        mesh=_mesh(),
        compiler_params=pltpu.CompilerParams(use_tc_tiling_on_sc=False),
        out_type=jax.ShapeDtypeStruct((NC, N_PAD, d), jnp.float32),
        scratch_types=[
            pltpu.VMEM((NCHUNK, C), jnp.int32),
            pltpu.VMEM((NCHUNK, C), jnp.int32),
            pltpu.VMEM((C, d), jnp.float32),
            pltpu.VMEM((C, d), jnp.float32),
            pltpu.VMEM((C, d), jnp.float32),
            pltpu.VMEM_SHARED((N_PAD, d), jnp.float32),
            pltpu.SemaphoreType.DMA,
            pltpu.SemaphoreType.DMA,
        ],
    )
    def k(h_h, src_h, dst_h, zeros_h, out_h,
          src_v, dst_v, gb0, gb1, bounce_v, acc_sh, s0, s1):
        cid = lax.axis_index("c")
        sid = lax.axis_index("s")
        wid = sid * NC + cid
        pltpu.sync_copy(zeros_h, bounce_v)

        def zrow(i, carry):
            pltpu.sync_copy(bounce_v, acc_sh.at[pl.ds(sid * RPT + i * C, C)])
            return carry

        lax.fori_loop(0, RPT // C, zrow, 0)
        pltpu.sync_copy(src_h.at[wid], src_v)
        pltpu.sync_copy(dst_h.at[wid], dst_v)
        plsc.subcore_barrier()

        cp0 = pltpu.async_copy(h_h.at[src_v.at[0]], gb0, s0)

        def body(jj, carry):
            j0 = jj * 2
            pltpu.async_copy(h_h.at[src_v.at[j0 + 1]], gb1, s1)
            pltpu.make_async_copy(h_h.at[src_v.at[j0]], gb0, s0).wait()
            pltpu.sync_copy(gb0, acc_sh.at[dst_v.at[j0]], add=True)
            pltpu.async_copy(h_h.at[src_v.at[j0 + 2]], gb0, s0)
            pltpu.make_async_copy(h_h.at[src_v.at[j0 + 1]], gb1, s1).wait()
            pltpu.sync_copy(gb1, acc_sh.at[dst_v.at[j0 + 1]], add=True)
            return carry

        lax.fori_loop(0, (NCHUNK - 2) // 2, body, 0)
        pltpu.async_copy(h_h.at[src_v.at[NCHUNK - 1]], gb1, s1)
        pltpu.make_async_copy(h_h.at[src_v.at[NCHUNK - 2]], gb0, s0).wait()
        pltpu.sync_copy(gb0, acc_sh.at[dst_v.at[NCHUNK - 2]], add=True)
        pltpu.make_async_copy(h_h.at[src_v.at[NCHUNK - 1]], gb1, s1).wait()
        pltpu.sync_copy(gb1, acc_sh.at[dst_v.at[NCHUNK - 1]], add=True)
        del cp0
        plsc.subcore_barrier()

        def wout(i, carry):
            pltpu.sync_copy(acc_sh.at[pl.ds(sid * RPT + i * C, C)], bounce_v)
            pltpu.sync_copy(bounce_v, out_h.at[cid, pl.ds(sid * RPT + i * C, C)])
            return carry

        lax.fori_loop(0, RPT // C, wout, 0)

    return k(h, src3, dst3, zeros_blk)


def _sc_agg_spmem(h, src3, dst3, zeros_blk, d):
    TR = 625

    @functools.partial(
        pl.kernel,
        mesh=_mesh(),
        compiler_params=pltpu.CompilerParams(use_tc_tiling_on_sc=False),
        out_type=jax.ShapeDtypeStruct((NC, N_PAD, d), jnp.float32),
        scratch_types=[
            pltpu.VMEM((NCHUNK, C), jnp.int32),
            pltpu.VMEM((NCHUNK, C), jnp.int32),
            pltpu.VMEM((C, d), jnp.float32),
            pltpu.VMEM((C, d), jnp.float32),
            pltpu.VMEM((C, d), jnp.float32),
            pltpu.VMEM_SHARED((N, d), jnp.float32),
            pltpu.VMEM_SHARED((N_PAD, d), jnp.float32),
            pltpu.SemaphoreType.DMA,
            pltpu.SemaphoreType.DMA,
        ],
    )
    def k(h_h, src_h, dst_h, zeros_h, out_h,
          src_v, dst_v, gb0, gb1, bounce_v, tab_sh, acc_sh, s0, s1):
        cid = lax.axis_index("c")
        sid = lax.axis_index("s")
        wid = sid * NC + cid
        pltpu.sync_copy(h_h.at[pl.ds(sid * TR, TR)], tab_sh.at[pl.ds(sid * TR, TR)])
        pltpu.sync_copy(zeros_h, bounce_v)

        def zrow(i, carry):
            pltpu.sync_copy(bounce_v, acc_sh.at[pl.ds(sid * RPT + i * C, C)])
            return carry

        lax.fori_loop(0, RPT // C, zrow, 0)
        pltpu.sync_copy(src_h.at[wid], src_v)
        pltpu.sync_copy(dst_h.at[wid], dst_v)
        plsc.subcore_barrier()

        cp0 = pltpu.async_copy(tab_sh.at[src_v.at[0]], gb0, s0)

        def body(jj, carry):
            j0 = jj * 2
            pltpu.async_copy(tab_sh.at[src_v.at[j0 + 1]], gb1, s1)
            pltpu.make_async_copy(tab_sh.at[src_v.at[j0]], gb0, s0).wait()
            pltpu.sync_copy(gb0, acc_sh.at[dst_v.at[j0]], add=True)
            pltpu.async_copy(tab_sh.at[src_v.at[j0 + 2]], gb0, s0)
            pltpu.make_async_copy(tab_sh.at[src_v.at[j0 + 1]], gb1, s1).wait()
            pltpu.sync_copy(gb1, acc_sh.at[dst_v.at[j0 + 1]], add=True)
            return carry

        lax.fori_loop(0, (NCHUNK - 2) // 2, body, 0)
        pltpu.async_copy(tab_sh.at[src_v.at[NCHUNK - 1]], gb1, s1)
        pltpu.make_async_copy(tab_sh.at[src_v.at[NCHUNK - 2]], gb0, s0).wait()
        pltpu.sync_copy(gb0, acc_sh.at[dst_v.at[NCHUNK - 2]], add=True)
        pltpu.make_async_copy(tab_sh.at[src_v.at[NCHUNK - 1]], gb1, s1).wait()
        pltpu.sync_copy(gb1, acc_sh.at[dst_v.at[NCHUNK - 1]], add=True)
        del cp0
        plsc.subcore_barrier()

        def wout(i, carry):
            pltpu.sync_copy(acc_sh.at[pl.ds(sid * RPT + i * C, C)], bounce_v)
            pltpu.sync_copy(bounce_v, out_h.at[cid, pl.ds(sid * RPT + i * C, C)])
            return carry

        lax.fori_loop(0, RPT // C, wout, 0)

    return k(h, src3, dst3, zeros_blk)


def _tc1a(x, bn_gamma, bn_beta, W1):
    def body(x_ref, g_ref, b_ref, w_ref, h_ref):
        xv = x_ref[...]
        mean = jnp.mean(xv, axis=0, keepdims=True)
        xc = xv - mean
        var = jnp.mean(xc * xc, axis=0, keepdims=True)
        xn = xc * lax.rsqrt(var + EPS) * g_ref[...] + b_ref[...]
        h_ref[...] = jnp.dot(xn, w_ref[...].T, preferred_element_type=jnp.float32)

    return pl.pallas_call(
        body,
        out_shape=jax.ShapeDtypeStruct((N, D_H), jnp.float32),
    )(x, bn_gamma, bn_beta, W1)


def _tc1b(h1, degT):
    def body(h_ref, deg_ref, hp_ref, dis_ref):
        deg = deg_ref[:, 0:1] + deg_ref[:, 1:2] + 1.0
        dis = lax.rsqrt(deg)
        hp_ref[...] = h_ref[...] * dis
        dis_ref[...] = dis

    return pl.pallas_call(
        body,
        out_shape=(
            jax.ShapeDtypeStruct((N, D_H), jnp.float32),
            jax.ShapeDtypeStruct((N, 1), jnp.float32),
        ),
    )(h1, degT)


def _tc2(A1, h1p, dis, b1, ln_gamma, ln_beta, W2):
    def body(a_ref, h_ref, dis_ref, b1_ref, lng_ref, lnb_ref, w2_ref, o_ref):
        dis = dis_ref[...]
        agg = a_ref[0, :N, :] + a_ref[1, :N, :] + h_ref[...]
        h1 = jnp.maximum(agg * dis + b1_ref[...], 0.0)
        m = jnp.mean(h1)
        hc = h1 - m
        v = jnp.mean(hc * hc)
        ln = hc * lax.rsqrt(v + EPS) * lng_ref[...] + lnb_ref[...]
        h2 = jnp.dot(ln, w2_ref[...].T, preferred_element_type=jnp.float32)
        o_ref[...] = h2 * dis

    return pl.pallas_call(
        body,
        out_shape=jax.ShapeDtypeStruct((N, D_OUT), jnp.float32),
    )(A1, h1p, dis, b1, ln_gamma, ln_beta, W2)


def _tc3(A2, h2p, dis, b2):
    def body(a_ref, h_ref, dis_ref, b2_ref, o_ref):
        agg = a_ref[0, :N, :] + a_ref[1, :N, :] + h_ref[...]
        o_ref[...] = jnp.maximum(agg * dis_ref[...] + b2_ref[...], 0.0)

    return pl.pallas_call(
        body,
        out_shape=jax.ShapeDtypeStruct((N, D_OUT), jnp.float32),
    )(A2, h2p, dis, b2)


def kernel(x, edge_index, bn_gamma, bn_beta, W1, b1, ln_gamma, ln_beta, W2, b2):
    src = edge_index[0]
    dst = edge_index[1]
    pad = E_PAD - E
    src3 = jnp.concatenate([src, jnp.zeros((pad,), jnp.int32)]).reshape(NW, NCHUNK, C)
    dst3 = jnp.concatenate([dst, jnp.full((pad,), N, jnp.int32)]).reshape(NW, NCHUNK, C)
    ones_c = jnp.ones((C,), jnp.float32)
    zeros_rpt = jnp.zeros((RPT,), jnp.float32)
    zeros_h96 = jnp.zeros((C, D_H), jnp.float32)
    zeros_h64 = jnp.zeros((C, D_OUT), jnp.float32)

    deg_parts = _sc_deg(dst3, ones_c, zeros_rpt)
    degT = deg_parts.T[:N]
    h1 = _tc1a(x, bn_gamma, bn_beta, W1)
    h1p, dis = _tc1b(h1, degT)
    A1 = _sc_agg(h1p, src3, dst3, zeros_h96, D_H)
    h2p = _tc2(A1, h1p, dis, b1, ln_gamma, ln_beta, W2)
    A2 = _sc_agg(h2p, src3, dst3, zeros_h64, D_OUT)
    return _tc3(A2, h2p, dis, b2)


# speedup vs baseline: 17.8308x; 1.3706x over previous
import functools

import jax
import jax.numpy as jnp
from jax import lax
from jax.experimental import pallas as pl
from jax.experimental.pallas import tpu as pltpu
from jax.experimental.pallas import tpu_sc as plsc

N = 10000
E = 320000
D_IN = 128
D_H = 96
D_OUT = 64
EPS = 1e-5

NC = 2
NS = 16
NW = NC * NS
C = 128
NCHUNK = 80
EPW = NCHUNK * C
E_PAD = NW * EPW
N_PAD = 10240
RPT = N_PAD // NS
NP_AGG = 10016
RPT_AGG = NP_AGG // NS


def _mesh():
    return plsc.VectorSubcoreMesh(core_axis_name="c", subcore_axis_name="s")


def _sc_deg(dst3, ones_c, zeros_rpt):
    @functools.partial(
        pl.kernel,
        mesh=_mesh(),
        compiler_params=pltpu.CompilerParams(use_tc_tiling_on_sc=False),
        out_type=jax.ShapeDtypeStruct((NC, N_PAD), jnp.float32),
        scratch_types=[
            pltpu.VMEM((NCHUNK, C), jnp.int32),
            pltpu.VMEM((C,), jnp.float32),
            pltpu.VMEM((RPT,), jnp.float32),
            pltpu.VMEM_SHARED((N_PAD,), jnp.float32),
        ],
    )
    def k(dst_h, ones_h, zeros_h, out_h, dst_v, ones_v, bounce_v, acc_sh):
        cid = lax.axis_index("c")
        sid = lax.axis_index("s")
        wid = sid * NC + cid
        pltpu.sync_copy(ones_h, ones_v)
        pltpu.sync_copy(zeros_h, bounce_v)
        pltpu.sync_copy(bounce_v, acc_sh.at[pl.ds(sid * RPT, RPT)])
        pltpu.sync_copy(dst_h.at[wid], dst_v)
        plsc.subcore_barrier()

        def body(j, carry):
            pltpu.sync_copy(ones_v, acc_sh.at[dst_v.at[j]], add=True)
            return carry

        lax.fori_loop(0, NCHUNK, body, 0)
        plsc.subcore_barrier()
        pltpu.sync_copy(acc_sh.at[pl.ds(sid * RPT, RPT)], bounce_v)
        pltpu.sync_copy(bounce_v, out_h.at[cid, pl.ds(sid * RPT, RPT)])

    return k(dst3, ones_c, zeros_rpt)


def _sc_agg(h, src3, dst3, zeros_blk, d):
    @functools.partial(
        pl.kernel,
        mesh=_mesh(),
        compiler_params=pltpu.CompilerParams(use_tc_tiling_on_sc=False),
        out_type=jax.ShapeDtypeStruct((NC, NP_AGG, d), jnp.float32),
        scratch_types=[
            pltpu.VMEM((NCHUNK, C), jnp.int32),
            pltpu.VMEM((NCHUNK, C), jnp.int32),
            [pltpu.VMEM((C, d), jnp.float32)] * 4,
            pltpu.VMEM_SHARED((NP_AGG, d), jnp.float32),
            [pltpu.SemaphoreType.DMA] * 4,
            [pltpu.SemaphoreType.DMA] * 4,
        ],
    )
    def k(h_h, src_h, dst_h, zeros_h, out_h,
          src_v, dst_v, gb, acc_sh, sg, ss):
        cid = lax.axis_index("c")
        sid = lax.axis_index("s")
        wid = sid * NC + cid
        pltpu.sync_copy(zeros_h, acc_sh.at[pl.ds(sid * RPT_AGG, RPT_AGG)])
        pltpu.sync_copy(src_h.at[wid], src_v)
        pltpu.sync_copy(dst_h.at[wid], dst_v)
        plsc.subcore_barrier()

        def gather(c, slot):
            pltpu.async_copy(h_h.at[src_v.at[c]], gb[slot], sg[slot])

        def wait_gather(c, slot):
            pltpu.make_async_copy(h_h.at[src_v.at[c]], gb[slot], sg[slot]).wait()

        def scatter(j, slot):
            pltpu.async_copy(gb[slot], acc_sh.at[dst_v.at[j]], ss[slot], add=True)

        def wait_scatter(j, slot):
            pltpu.make_async_copy(gb[slot], acc_sh.at[dst_v.at[j]], ss[slot]).wait()

        gather(0, 0)
        gather(1, 1)
        for k_ in range(4):
            if k_ >= 2:
                wait_scatter(k_ - 2, k_ - 2)
            gather(k_ + 2, (k_ + 2) % 4)
            wait_gather(k_, k_)
            scatter(k_, k_)

        def body(g, carry):
            b = g * 4
            for k_ in range(4):
                wait_scatter(b + k_ - 2, (k_ + 2) % 4)
                gather(b + k_ + 2, (k_ + 2) % 4)
                wait_gather(b + k_, k_)
                scatter(b + k_, k_)
            return carry

        lax.fori_loop(1, NCHUNK // 4 - 1, body, 0)
        b = NCHUNK - 4
        for k_ in range(4):
            wait_scatter(b + k_ - 2, (k_ + 2) % 4)
            if k_ < 2:
                gather(b + k_ + 2, (k_ + 2) % 4)
            wait_gather(b + k_, k_)
            scatter(b + k_, k_)
        wait_scatter(NCHUNK - 2, 2)
        wait_scatter(NCHUNK - 1, 3)
        plsc.subcore_barrier()
        pltpu.sync_copy(acc_sh.at[pl.ds(sid * RPT_AGG, RPT_AGG)],
                        out_h.at[cid, pl.ds(sid * RPT_AGG, RPT_AGG)])

    return k(h, src3, dst3, zeros_blk)


C2 = 112
NCHUNK2 = 90
E_PAD2 = NW * NCHUNK2 * C2


def _sc_agg2sp(h, src3, dst3, zeros_blk):
    d = D_OUT
    TR = N // NS

    @functools.partial(
        pl.kernel,
        mesh=_mesh(),
        compiler_params=pltpu.CompilerParams(use_tc_tiling_on_sc=False),
        out_type=jax.ShapeDtypeStruct((NC, NP_AGG, d), jnp.float32),
        scratch_types=[
            pltpu.VMEM((NCHUNK2, C2), jnp.int32),
            pltpu.VMEM((NCHUNK2, C2), jnp.int32),
            [pltpu.VMEM((C2, d), jnp.float32)] * 4,
            pltpu.VMEM_SHARED((N, d), jnp.float32),
            pltpu.VMEM_SHARED((NP_AGG, d), jnp.float32),
            [pltpu.SemaphoreType.DMA] * 4,
            [pltpu.SemaphoreType.DMA] * 4,
        ],
    )
    def k(h_h, src_h, dst_h, zeros_h, out_h,
          src_v, dst_v, gb, tab_sh, acc_sh, sg, ss):
        cid = lax.axis_index("c")
        sid = lax.axis_index("s")
        wid = sid * NC + cid
        pltpu.sync_copy(h_h.at[pl.ds(sid * TR, TR)], tab_sh.at[pl.ds(sid * TR, TR)])
        pltpu.sync_copy(zeros_h, acc_sh.at[pl.ds(sid * RPT_AGG, RPT_AGG)])
        pltpu.sync_copy(src_h.at[wid], src_v)
        pltpu.sync_copy(dst_h.at[wid], dst_v)
        plsc.subcore_barrier()

        def gather(c, slot):
            pltpu.async_copy(tab_sh.at[src_v.at[c]], gb[slot], sg[slot])

        def wait_gather(c, slot):
            pltpu.make_async_copy(tab_sh.at[src_v.at[c]], gb[slot], sg[slot]).wait()

        def scatter(j, slot):
            pltpu.async_copy(gb[slot], acc_sh.at[dst_v.at[j]], ss[slot], add=True)

        def wait_scatter(j, slot):
            pltpu.make_async_copy(gb[slot], acc_sh.at[dst_v.at[j]], ss[slot]).wait()

        gather(0, 0)
        gather(1, 1)
        for k_ in range(4):
            if k_ >= 2:
                wait_scatter(k_ - 2, k_ - 2)
            gather(k_ + 2, (k_ + 2) % 4)
            wait_gather(k_, k_)
            scatter(k_, k_)

        def body(g, carry):
            b = g * 4
            for k_ in range(4):
                wait_scatter(b + k_ - 2, (k_ + 2) % 4)
                gather(b + k_ + 2, (k_ + 2) % 4)
                wait_gather(b + k_, k_)
                scatter(b + k_, k_)
            return carry

        lax.fori_loop(1, NCHUNK2 // 4, body, 0)
        wait_scatter(NCHUNK2 - 4, 2)
        wait_gather(NCHUNK2 - 2, 0)
        scatter(NCHUNK2 - 2, 0)
        wait_scatter(NCHUNK2 - 3, 3)
        wait_gather(NCHUNK2 - 1, 1)
        scatter(NCHUNK2 - 1, 1)
        wait_scatter(NCHUNK2 - 2, 0)
        wait_scatter(NCHUNK2 - 1, 1)
        plsc.subcore_barrier()
        pltpu.sync_copy(acc_sh.at[pl.ds(sid * RPT_AGG, RPT_AGG)],
                        out_h.at[cid, pl.ds(sid * RPT_AGG, RPT_AGG)])

    return k(h, src3, dst3, zeros_blk)


def _tc1a(x, bn_gamma, bn_beta, W1):
    def body(x_ref, g_ref, b_ref, w_ref, h_ref):
        xv = x_ref[...]
        mean = jnp.mean(xv, axis=0, keepdims=True)
        xc = xv - mean
        var = jnp.mean(xc * xc, axis=0, keepdims=True)
        xn = xc * lax.rsqrt(var + EPS) * g_ref[...] + b_ref[...]
        h_ref[...] = jnp.dot(xn, w_ref[...].T, preferred_element_type=jnp.float32)

    return pl.pallas_call(
        body,
        out_shape=jax.ShapeDtypeStruct((N, D_H), jnp.float32),
    )(x, bn_gamma, bn_beta, W1)


def _tc1b(h1, degT):
    def body(h_ref, deg_ref, hp_ref, dis_ref):
        deg = deg_ref[:, 0:1] + deg_ref[:, 1:2] + 1.0
        dis = lax.rsqrt(deg)
        hp_ref[...] = h_ref[...] * dis
        dis_ref[...] = dis

    return pl.pallas_call(
        body,
        out_shape=(
            jax.ShapeDtypeStruct((N, D_H), jnp.float32),
            jax.ShapeDtypeStruct((N, 1), jnp.float32),
        ),
    )(h1, degT)


def _tc2(A1, h1p, dis, b1, ln_gamma, ln_beta, W2):
    def body(a_ref, h_ref, dis_ref, b1_ref, lng_ref, lnb_ref, w2_ref, o_ref):
        dis = dis_ref[...]
        agg = a_ref[0, :N, :] + a_ref[1, :N, :] + h_ref[...]
        h1 = jnp.maximum(agg * dis + b1_ref[...], 0.0)
        m = jnp.mean(h1)
        hc = h1 - m
        v = jnp.mean(hc * hc)
        ln = hc * lax.rsqrt(v + EPS) * lng_ref[...] + lnb_ref[...]
        h2 = jnp.dot(ln, w2_ref[...].T, preferred_element_type=jnp.float32)
        o_ref[...] = h2 * dis

    return pl.pallas_call(
        body,
        out_shape=jax.ShapeDtypeStruct((N, D_OUT), jnp.float32),
    )(A1, h1p, dis, b1, ln_gamma, ln_beta, W2)


def _tc3(A2, h2p, dis, b2):
    def body(a_ref, h_ref, dis_ref, b2_ref, o_ref):
        agg = a_ref[0, :N, :] + a_ref[1, :N, :] + h_ref[...]
        o_ref[...] = jnp.maximum(agg * dis_ref[...] + b2_ref[...], 0.0)

    return pl.pallas_call(
        body,
        out_shape=jax.ShapeDtypeStruct((N, D_OUT), jnp.float32),
    )(A2, h2p, dis, b2)


def kernel(x, edge_index, bn_gamma, bn_beta, W1, b1, ln_gamma, ln_beta, W2, b2):
    src = edge_index[0]
    dst = edge_index[1]
    pad = E_PAD - E
    src3 = jnp.concatenate([src, jnp.zeros((pad,), jnp.int32)]).reshape(NW, NCHUNK, C)
    dst3 = jnp.concatenate([dst, jnp.full((pad,), N, jnp.int32)]).reshape(NW, NCHUNK, C)
    pad2 = E_PAD2 - E
    src3b = jnp.concatenate([src, jnp.zeros((pad2,), jnp.int32)]).reshape(NW, NCHUNK2, C2)
    dst3b = jnp.concatenate([dst, jnp.full((pad2,), N, jnp.int32)]).reshape(NW, NCHUNK2, C2)
    ones_c = jnp.ones((C,), jnp.float32)
    zeros_rpt = jnp.zeros((RPT,), jnp.float32)
    zeros_h96 = jnp.zeros((RPT_AGG, D_H), jnp.float32)
    zeros_h64 = jnp.zeros((RPT_AGG, D_OUT), jnp.float32)

    deg_parts = _sc_deg(dst3, ones_c, zeros_rpt)
    degT = deg_parts.T[:N]
    h1 = _tc1a(x, bn_gamma, bn_beta, W1)
    h1p, dis = _tc1b(h1, degT)
    A1 = _sc_agg(h1p, src3, dst3, zeros_h96, D_H)
    h2p = _tc2(A1, h1p, dis, b1, ln_gamma, ln_beta, W2)
    A2 = _sc_agg2sp(h2p, src3b, dst3b, zeros_h64)
    return _tc3(A2, h2p, dis, b2)
